# baseline (device time: 233457 ns/iter reference)
import jax
import jax.numpy as jnp
from jax import lax
from jax.experimental import pallas as pl
from jax.experimental.pallas import tpu as pltpu

N_DEV = 4
N_HOPS = N_DEV - 1


def _all_reduce_body(
    x_ref, out_ref,
    acc_ref, comm_cw, comm_ccw,
    send_cw, recv_cw, send_ccw, recv_ccw,
    ag_send_cw, ag_recv_cw, ag_send_ccw, ag_recv_ccw,
):
    m, n = x_ref.shape
    chunk = m // N_DEV
    half = n // 2
    cw_cols = slice(0, half)
    ccw_cols = slice(half, n)

    d = lax.axis_index("i")
    left = lax.rem(d + N_DEV - 1, N_DEV)
    right = lax.rem(d + 1, N_DEV)

    barrier_sem = pltpu.get_barrier_semaphore()
    for nbr in [left, right]:
        pl.semaphore_signal(
            barrier_sem, inc=1,
            device_id=(nbr,), device_id_type=pl.DeviceIdType.MESH,
        )
    pl.semaphore_wait(barrier_sem, 2)

    acc_ref[...] = x_ref[...].astype(jnp.bfloat16)

    def row(idx):
        return pl.ds(idx * chunk, chunk)

    for h in range(N_HOPS):
        cw_send = lax.rem(d + (N_DEV - h), N_DEV)
        cw_recv = lax.rem(d + (2 * N_DEV - h - 1), N_DEV)
        ccw_send = lax.rem(d + h, N_DEV)
        ccw_recv = lax.rem(d + h + 1, N_DEV)

        rdma_cw = pltpu.make_async_remote_copy(
            src_ref=acc_ref.at[row(cw_send), cw_cols],
            dst_ref=comm_cw.at[h],
            send_sem=send_cw.at[h],
            recv_sem=recv_cw.at[h],
            device_id=(right,),
            device_id_type=pl.DeviceIdType.MESH,
        )
        rdma_ccw = pltpu.make_async_remote_copy(
            src_ref=acc_ref.at[row(ccw_send), ccw_cols],
            dst_ref=comm_ccw.at[h],
            send_sem=send_ccw.at[h],
            recv_sem=recv_ccw.at[h],
            device_id=(left,),
            device_id_type=pl.DeviceIdType.MESH,
        )
        rdma_cw.start()
        rdma_ccw.start()
        rdma_cw.wait()
        rdma_ccw.wait()
        acc_ref[row(cw_recv), cw_cols] = (
            acc_ref[row(cw_recv), cw_cols] + comm_cw[h]
        )
        acc_ref[row(ccw_recv), ccw_cols] = (
            acc_ref[row(ccw_recv), ccw_cols] + comm_ccw[h]
        )

    for g in range(N_HOPS):
        cw_send = lax.rem(d + (N_DEV + 1 - g), N_DEV)
        ccw_send = lax.rem(d + (N_DEV - 1 + g), N_DEV)

        rdma_cw = pltpu.make_async_remote_copy(
            src_ref=acc_ref.at[row(cw_send), cw_cols],
            dst_ref=acc_ref.at[row(cw_send), cw_cols],
            send_sem=ag_send_cw.at[g],
            recv_sem=ag_recv_cw.at[g],
            device_id=(right,),
            device_id_type=pl.DeviceIdType.MESH,
        )
        rdma_ccw = pltpu.make_async_remote_copy(
            src_ref=acc_ref.at[row(ccw_send), ccw_cols],
            dst_ref=acc_ref.at[row(ccw_send), ccw_cols],
            send_sem=ag_send_ccw.at[g],
            recv_sem=ag_recv_ccw.at[g],
            device_id=(left,),
            device_id_type=pl.DeviceIdType.MESH,
        )
        rdma_cw.start()
        rdma_ccw.start()
        rdma_cw.wait()
        rdma_ccw.wait()

    out_ref[...] = acc_ref[...].astype(jnp.float32)


def _ring_all_reduce(x):
    m, n = x.shape
    chunk = m // N_DEV
    half = n // 2
    sem = pltpu.SemaphoreType.DMA((N_HOPS,))
    return pl.pallas_call(
        _all_reduce_body,
        out_shape=jax.ShapeDtypeStruct((m, n), x.dtype),
        in_specs=[pl.BlockSpec(memory_space=pltpu.VMEM)],
        out_specs=pl.BlockSpec(memory_space=pltpu.VMEM),
        scratch_shapes=[
            pltpu.VMEM((m, n), jnp.bfloat16),
            pltpu.VMEM((N_HOPS, chunk, half), jnp.bfloat16),
            pltpu.VMEM((N_HOPS, chunk, half), jnp.bfloat16),
            sem, sem, sem, sem,
            sem, sem, sem, sem,
        ],
        compiler_params=pltpu.CompilerParams(
            collective_id=0,
            vmem_limit_bytes=100 * 1024 * 1024,
        ),
    )(x)


def _matmul_body(dy_ref, w_ref, out_ref):
    out_ref[...] = lax.dot_general(
        dy_ref[...],
        w_ref[...],
        (((1,), (1,)), ((), ())),
        preferred_element_type=jnp.float32,
    )


def _pallas_partial_matmul(dy, W):
    m, k = dy.shape
    n, _ = W.shape
    bm = 512
    bn = 256
    return pl.pallas_call(
        _matmul_body,
        grid=(m // bm, n // bn),
        in_specs=[
            pl.BlockSpec((bm, k), lambda c, j: (c, 0)),
            pl.BlockSpec((bn, k), lambda c, j: (j, 0)),
        ],
        out_specs=pl.BlockSpec((bm, bn), lambda c, j: (c, j)),
        out_shape=jax.ShapeDtypeStruct((m, n), jnp.float32),
        compiler_params=pltpu.CompilerParams(
            dimension_semantics=("arbitrary", "arbitrary"),
            vmem_limit_bytes=100 * 1024 * 1024,
        ),
    )(dy, W)


def kernel(dy, W):
    partial = _pallas_partial_matmul(dy, W)
    return _ring_all_reduce(partial)
